# baseline (speedup 1.0000x reference)
import functools

import jax
import jax.numpy as jnp
from jax import lax
from jax.experimental import pallas as pl
from jax.experimental.pallas import tpu as pltpu
from jax.experimental.pallas import tpu_sc as plsc

NC = 2
NS = 16
NW = NC * NS
L = 16
ND = 4


def _pack_pairs_kernel(blk_ref, out_ref):
    blk = blk_ref[...]
    nxt = jnp.concatenate([blk[:, 1:], blk[:, :1]], axis=1)
    lo = lax.bitcast_convert_type(blk.astype(jnp.bfloat16), jnp.uint16)
    hi = lax.bitcast_convert_type(nxt.astype(jnp.bfloat16), jnp.uint16)
    packed = lo.astype(jnp.uint32) | (hi.astype(jnp.uint32) << 16)
    out_ref[...] = lax.bitcast_convert_type(packed, jnp.int32)


def _pack_pairs(table2d):
    h, w = table2d.shape
    r = 128
    return pl.pallas_call(
        _pack_pairs_kernel,
        out_shape=jax.ShapeDtypeStruct((h, w), jnp.int32),
        grid=(h // r,),
        in_specs=[pl.BlockSpec((r, w), lambda i: (i, 0))],
        out_specs=pl.BlockSpec((r, w), lambda i: (i, 0)),
    )(table2d)


def _make_sc_gather(n, h, w):
    per_w = n // NW
    t = 2048
    nb = per_w // t
    mesh = plsc.VectorSubcoreMesh(core_axis_name="c", subcore_axis_name="s")

    ring = lambda shp, dt: [pltpu.VMEM(shp, dt) for _ in range(ND)]

    @functools.partial(
        pl.kernel,
        mesh=mesh,
        out_type=(jax.ShapeDtypeStruct((n,), jnp.int32),
                  jax.ShapeDtypeStruct((n,), jnp.int32)),
        scratch_types=(
            [pltpu.VMEM((4, L), jnp.float32)]
            + ring((t,), jnp.float32) + ring((t,), jnp.float32)
            + ring((2 * t,), jnp.int32) + ring((2 * t,), jnp.int32)
            + [pltpu.SemaphoreType.DMA] * ND
        ),
    )
    def kern(x1_hbm, x2_hbm, qa_hbm, params_hbm, v1_hbm, v2_hbm,
             params_v, *sc):
        x1s, x2s = sc[0:ND], sc[ND:2 * ND]
        idxs, valss = sc[2 * ND:3 * ND], sc[3 * ND:4 * ND]
        sems = sc[4 * ND:5 * ND]

        cid = lax.axis_index("c")
        sid = lax.axis_index("s")
        wid = sid * NC + cid
        base_w = wid * per_w

        pltpu.sync_copy(params_hbm, params_v)
        sy = params_v[0]
        sx = params_v[1]
        oy = params_v[2]
        ox = params_v[3]

        def load_inputs(b, p):
            off = base_w + b * t
            pltpu.sync_copy(x1_hbm.at[pl.ds(off, t)], x1s[p])
            pltpu.sync_copy(x2_hbm.at[pl.ds(off, t)], x2s[p])

        def compute_idx(p):
            x1_v, x2_v, idx_v = x1s[p], x2s[p], idxs[p]

            def body(j, carry):
                s = j * L
                x1 = x1_v[pl.ds(s, L)]
                x2 = x2_v[pl.ds(s, L)]
                qy = jnp.maximum(x1 * sy + oy, 0.0)
                qx = jnp.maximum(x2 * sx + ox, 0.0)
                fy = jnp.minimum(qy.astype(jnp.int32), h - 2)
                fx = jnp.minimum(qx.astype(jnp.int32), w - 2)
                lin = fy * w + fx
                idx_v[pl.ds(s, L)] = lin
                idx_v[pl.ds(t + s, L)] = lin + w
                return carry

            lax.fori_loop(0, t // L, body, 0, unroll=8)

        def start_gather(p):
            return pltpu.async_copy(qa_hbm.at[idxs[p]], valss[p], sems[p])

        def store_vals(b, p):
            off = base_w + b * t
            pltpu.sync_copy(valss[p].at[pl.ds(0, t)], v1_hbm.at[pl.ds(off, t)])
            pltpu.sync_copy(valss[p].at[pl.ds(t, t)], v2_hbm.at[pl.ds(off, t)])

        handles = {}
        for b in range(nb):
            p = b % ND
            if b >= ND:
                with jax.named_scope("gather_wait"):
                    handles[b - ND].wait()
                with jax.named_scope("store_vals"):
                    store_vals(b - ND, p)
            with jax.named_scope("load_inputs"):
                load_inputs(b, p)
            with jax.named_scope("compute_idx"):
                compute_idx(p)
            handles[b] = start_gather(p)
        for b in range(nb - ND, nb):
            with jax.named_scope("gather_wait"):
                handles[b].wait()
            with jax.named_scope("store_vals"):
                store_vals(b, b % ND)

    return kern


def _combine_kernel(h, w, params_ref, x1_ref, x2_ref, v1_ref, v2_ref,
                    out_ref):
    sy = params_ref[0]
    sx = params_ref[1]
    oy = params_ref[2]
    ox = params_ref[3]
    x1 = x1_ref[...]
    x2 = x2_ref[...]
    v1 = v1_ref[...]
    v2 = v2_ref[...]
    qy = jnp.maximum(x1 * sy + oy, 0.0)
    qx = jnp.maximum(x2 * sx + ox, 0.0)
    fy = jnp.minimum(jnp.floor(qy), float(h - 2))
    fx = jnp.minimum(jnp.floor(qx), float(w - 2))
    ay = jnp.minimum(qy - fy, 1.0)
    ax = jnp.minimum(qx - fx, 1.0)
    himask = jnp.int32(-65536)
    tl = lax.bitcast_convert_type(v1 << 16, jnp.float32)
    tr = lax.bitcast_convert_type(v1 & himask, jnp.float32)
    bl = lax.bitcast_convert_type(v2 << 16, jnp.float32)
    br = lax.bitcast_convert_type(v2 & himask, jnp.float32)
    top = ax * (tr - tl) + tl
    bot = ax * (br - bl) + bl
    out_ref[...] = ay * (bot - top) + top


def _combine(h, w, params, x1, x2, v1, v2):
    n = x1.shape[0]
    rows, cols = n // 128, 128
    blk = 512
    spec = pl.BlockSpec((blk, cols), lambda i: (i, 0))
    return pl.pallas_call(
        functools.partial(_combine_kernel, h, w),
        out_shape=jax.ShapeDtypeStruct((rows, cols), jnp.float32),
        grid=(rows // blk,),
        in_specs=[pl.BlockSpec(memory_space=pltpu.SMEM),
                  spec, spec, spec, spec],
        out_specs=spec,
    )(params, x1.reshape(rows, cols), x2.reshape(rows, cols),
      v1.reshape(rows, cols), v2.reshape(rows, cols))


def kernel(inputs, grid, bounds):
    n = inputs.shape[0]
    _, h, w, _ = grid.shape
    scale = (jnp.array([h, w], jnp.float32) - 1.0) / (bounds[1] - bounds[0])
    off = -bounds[0] * scale
    params_sc = jnp.broadcast_to(
        jnp.concatenate([scale, off]).reshape(4, 1), (4, L)
    ).astype(jnp.float32)
    params_tc = jnp.concatenate([scale, off]).astype(jnp.float32)
    planes = inputs.T
    qa = _pack_pairs(grid.reshape(h, w)).reshape(-1)
    v1, v2 = _make_sc_gather(n, h, w)(planes[0], planes[1], qa, params_sc)
    out = _combine(h, w, params_tc, planes[0], planes[1], v1, v2)
    return out.reshape(n, 1)


# speedup vs baseline: 1.2287x; 1.2287x over previous
import functools

import jax
import jax.numpy as jnp
from jax import lax
from jax.experimental import pallas as pl
from jax.experimental.pallas import tpu as pltpu
from jax.experimental.pallas import tpu_sc as plsc

NC = 2
NS = 16
NW = NC * NS
L = 16
ND = 4


def _pack_pairs_kernel(blk_ref, out_ref):
    blk = blk_ref[...]
    col0_up = jnp.concatenate([blk[1:, :1], blk[:1, :1]], axis=0)
    nxt = jnp.concatenate([blk[:, 1:], col0_up], axis=1)
    lo = lax.bitcast_convert_type(blk.astype(jnp.bfloat16), jnp.uint16)
    hi = lax.bitcast_convert_type(nxt.astype(jnp.bfloat16), jnp.uint16)
    packed = lo.astype(jnp.uint32) | (hi.astype(jnp.uint32) << 16)
    out_ref[...] = lax.bitcast_convert_type(packed, jnp.int32)


def _pack_pairs(t128):
    rows = t128.shape[0]
    r = 1024
    spec = pl.BlockSpec((r, 128), lambda i: (i, 0))
    return pl.pallas_call(
        _pack_pairs_kernel,
        out_shape=jax.ShapeDtypeStruct((rows, 128), jnp.int32),
        grid=(rows // r,),
        in_specs=[spec],
        out_specs=spec,
    )(t128)


def _index_kernel(h, w, params_ref, x1_ref, x2_ref, out_ref):
    sy = params_ref[0]
    sx = params_ref[1]
    oy = params_ref[2]
    ox = params_ref[3]
    qy = jnp.maximum(x1_ref[...] * sy + oy, 0.0)
    qx = jnp.maximum(x2_ref[...] * sx + ox, 0.0)
    fy = jnp.minimum(qy.astype(jnp.int32), h - 2)
    fx = jnp.minimum(qx.astype(jnp.int32), w - 2)
    out_ref[...] = fy * w + fx


def _index(h, w, params, x1r, x2r):
    rows = x1r.shape[0]
    blk = 1024
    spec = pl.BlockSpec((blk, 128), lambda i: (i, 0))
    return pl.pallas_call(
        functools.partial(_index_kernel, h, w),
        out_shape=jax.ShapeDtypeStruct((rows, 128), jnp.int32),
        grid=(rows // blk,),
        in_specs=[pl.BlockSpec(memory_space=pltpu.SMEM), spec, spec],
        out_specs=spec,
    )(params, x1r, x2r)


def _make_sc_gather(n, w):
    per_w = n // NW
    t = 2048
    nb = per_w // t
    mesh = plsc.VectorSubcoreMesh(core_axis_name="c", subcore_axis_name="s")

    ring = lambda shp, dt: [pltpu.VMEM(shp, dt) for _ in range(ND)]

    @functools.partial(
        pl.kernel,
        mesh=mesh,
        out_type=(jax.ShapeDtypeStruct((n,), jnp.int32),
                  jax.ShapeDtypeStruct((n,), jnp.int32)),
        scratch_types=(
            ring((2 * t,), jnp.int32) + ring((2 * t,), jnp.int32)
            + [pltpu.SemaphoreType.DMA] * ND
        ),
    )
    def kern(lin_hbm, qa_hbm, v1_hbm, v2_hbm, *sc):
        idxs, valss = sc[0:ND], sc[ND:2 * ND]
        sems = sc[2 * ND:3 * ND]

        cid = lax.axis_index("c")
        sid = lax.axis_index("s")
        wid = sid * NC + cid
        base_w = wid * per_w

        def load_idx(b, p):
            off = base_w + b * t
            idx_v = idxs[p]
            pltpu.sync_copy(lin_hbm.at[pl.ds(off, t)], idx_v.at[pl.ds(0, t)])

            def body(j, carry):
                s = j * L
                idx_v[pl.ds(t + s, L)] = idx_v[pl.ds(s, L)] + w
                return carry

            lax.fori_loop(0, t // L, body, 0, unroll=8)

        def start_gather(p):
            return pltpu.async_copy(qa_hbm.at[idxs[p]], valss[p], sems[p])

        def store_vals(b, p):
            off = base_w + b * t
            pltpu.sync_copy(valss[p].at[pl.ds(0, t)], v1_hbm.at[pl.ds(off, t)])
            pltpu.sync_copy(valss[p].at[pl.ds(t, t)], v2_hbm.at[pl.ds(off, t)])

        handles = {}
        for b in range(nb):
            p = b % ND
            if b >= ND:
                with jax.named_scope("gather_wait"):
                    handles[b - ND].wait()
                with jax.named_scope("store_vals"):
                    store_vals(b - ND, p)
            with jax.named_scope("load_idx"):
                load_idx(b, p)
            handles[b] = start_gather(p)
        for b in range(nb - ND, nb):
            with jax.named_scope("gather_wait"):
                handles[b].wait()
            with jax.named_scope("store_vals"):
                store_vals(b, b % ND)

    return kern


def _combine_kernel(h, w, params_ref, x1_ref, x2_ref, v1_ref, v2_ref,
                    out_ref):
    sy = params_ref[0]
    sx = params_ref[1]
    oy = params_ref[2]
    ox = params_ref[3]
    qy = jnp.maximum(x1_ref[...] * sy + oy, 0.0)
    qx = jnp.maximum(x2_ref[...] * sx + ox, 0.0)
    fy = jnp.minimum(jnp.floor(qy), float(h - 2))
    fx = jnp.minimum(jnp.floor(qx), float(w - 2))
    ay = jnp.minimum(qy - fy, 1.0)
    ax = jnp.minimum(qx - fx, 1.0)
    v1 = v1_ref[...]
    v2 = v2_ref[...]
    himask = jnp.int32(-65536)
    tl = lax.bitcast_convert_type(v1 << 16, jnp.float32)
    tr = lax.bitcast_convert_type(v1 & himask, jnp.float32)
    bl = lax.bitcast_convert_type(v2 << 16, jnp.float32)
    br = lax.bitcast_convert_type(v2 & himask, jnp.float32)
    top = ax * (tr - tl) + tl
    bot = ax * (br - bl) + bl
    out_ref[...] = ay * (bot - top) + top


def _combine(h, w, params, x1r, x2r, v1r, v2r):
    rows = x1r.shape[0]
    blk = 1024
    spec = pl.BlockSpec((blk, 128), lambda i: (i, 0))
    return pl.pallas_call(
        functools.partial(_combine_kernel, h, w),
        out_shape=jax.ShapeDtypeStruct((rows, 128), jnp.float32),
        grid=(rows // blk,),
        in_specs=[pl.BlockSpec(memory_space=pltpu.SMEM),
                  spec, spec, spec, spec],
        out_specs=spec,
    )(params, x1r, x2r, v1r, v2r)


def kernel(inputs, grid, bounds):
    n = inputs.shape[0]
    _, h, w, _ = grid.shape
    scale = (jnp.array([h, w], jnp.float32) - 1.0) / (bounds[1] - bounds[0])
    off = -bounds[0] * scale
    params = jnp.concatenate([scale, off]).astype(jnp.float32)
    rows = n // 128
    x1r = inputs[:, 0].reshape(rows, 128)
    x2r = inputs[:, 1].reshape(rows, 128)
    qa = _pack_pairs(grid.reshape(h * w // 128, 128))
    lin = _index(h, w, params, x1r, x2r)
    v1, v2 = _make_sc_gather(n, w)(lin.reshape(-1), qa.reshape(-1))
    out = _combine(h, w, params, x1r, x2r,
                   v1.reshape(rows, 128), v2.reshape(rows, 128))
    return out.reshape(n, 1)


# speedup vs baseline: 1.3037x; 1.0611x over previous
import functools

import jax
import jax.numpy as jnp
from jax import lax
from jax.experimental import pallas as pl
from jax.experimental.pallas import tpu as pltpu
from jax.experimental.pallas import tpu_sc as plsc

NC = 2
NS = 16
NW = NC * NS
L = 16
ND = 4


def _pack_pairs_kernel(blk_ref, out_ref):
    blk = blk_ref[...]
    col0_up = jnp.concatenate([blk[1:, :1], blk[:1, :1]], axis=0)
    nxt = jnp.concatenate([blk[:, 1:], col0_up], axis=1)
    lo = lax.bitcast_convert_type(blk.astype(jnp.bfloat16), jnp.uint16)
    hi = lax.bitcast_convert_type(nxt.astype(jnp.bfloat16), jnp.uint16)
    packed = lo.astype(jnp.uint32) | (hi.astype(jnp.uint32) << 16)
    out_ref[...] = lax.bitcast_convert_type(packed, jnp.int32)


def _pack_pairs(t128):
    rows = t128.shape[0]
    r = 2048
    spec = pl.BlockSpec((r, 128), lambda i: (i, 0))
    return pl.pallas_call(
        _pack_pairs_kernel,
        out_shape=jax.ShapeDtypeStruct((rows, 128), jnp.int32),
        grid=(rows // r,),
        in_specs=[spec],
        out_specs=spec,
    )(t128)


def _index_kernel(h, w, params_ref, x1_ref, x2_ref, out_ref):
    sy = params_ref[0]
    sx = params_ref[1]
    oy = params_ref[2]
    ox = params_ref[3]
    qy = jnp.maximum(x1_ref[...] * sy + oy, 0.0)
    qx = jnp.maximum(x2_ref[...] * sx + ox, 0.0)
    fy = jnp.minimum(qy.astype(jnp.int32), h - 2)
    fx = jnp.minimum(qx.astype(jnp.int32), w - 2)
    out_ref[...] = fy * w + fx


def _index(h, w, params, x1r, x2r):
    rows = x1r.shape[0]
    blk = 1024
    spec = pl.BlockSpec((blk, 128), lambda i: (i, 0))
    return pl.pallas_call(
        functools.partial(_index_kernel, h, w),
        out_shape=jax.ShapeDtypeStruct((rows, 128), jnp.int32),
        grid=(rows // blk,),
        in_specs=[pl.BlockSpec(memory_space=pltpu.SMEM), spec, spec],
        out_specs=spec,
    )(params, x1r, x2r)


def _make_sc_gather(n, w):
    per_w = n // NW
    t = 2048
    nb = per_w // t
    mesh = plsc.VectorSubcoreMesh(core_axis_name="c", subcore_axis_name="s")

    ring = lambda shp, dt: [pltpu.VMEM(shp, dt) for _ in range(ND)]

    @functools.partial(
        pl.kernel,
        mesh=mesh,
        out_type=(jax.ShapeDtypeStruct((n,), jnp.int32),
                  jax.ShapeDtypeStruct((n,), jnp.int32)),
        scratch_types=(
            [pltpu.VMEM((per_w,), jnp.int32)]
            + ring((2 * t,), jnp.int32) + ring((2 * t,), jnp.int32)
            + [pltpu.SemaphoreType.DMA] * ND
        ),
    )
    def kern(lin_hbm, qa_hbm, v1_hbm, v2_hbm, lin_v, *sc):
        idxs, valss = sc[0:ND], sc[ND:2 * ND]
        sems = sc[2 * ND:3 * ND]

        cid = lax.axis_index("c")
        sid = lax.axis_index("s")
        wid = sid * NC + cid
        base_w = wid * per_w

        def build_idx(b, p):
            idx_v = idxs[p]
            base = b * t

            def body(j, carry):
                s = j * L
                v = lin_v[pl.ds(base + s, L)]
                idx_v[pl.ds(s, L)] = v
                idx_v[pl.ds(t + s, L)] = v + w
                return carry

            lax.fori_loop(0, t // L, body, 0, unroll=8)

        def start_gather(p):
            return pltpu.async_copy(qa_hbm.at[idxs[p]], valss[p], sems[p])

        def store_vals(b, p):
            off = base_w + b * t
            pltpu.sync_copy(valss[p].at[pl.ds(0, t)], v1_hbm.at[pl.ds(off, t)])
            pltpu.sync_copy(valss[p].at[pl.ds(t, t)], v2_hbm.at[pl.ds(off, t)])

        with jax.named_scope("load_lin"):
            pltpu.sync_copy(lin_hbm.at[pl.ds(base_w, per_w)], lin_v)

        handles = {}
        for b in range(nb):
            p = b % ND
            if b >= ND:
                with jax.named_scope("gather_wait"):
                    handles[b - ND].wait()
                with jax.named_scope("store_vals"):
                    store_vals(b - ND, p)
            with jax.named_scope("build_idx"):
                build_idx(b, p)
            handles[b] = start_gather(p)
        for b in range(nb - ND, nb):
            with jax.named_scope("gather_wait"):
                handles[b].wait()
            with jax.named_scope("store_vals"):
                store_vals(b, b % ND)

    return kern


def _combine_kernel(h, w, params_ref, x1_ref, x2_ref, v1_ref, v2_ref,
                    out_ref):
    sy = params_ref[0]
    sx = params_ref[1]
    oy = params_ref[2]
    ox = params_ref[3]
    qy = jnp.maximum(x1_ref[...] * sy + oy, 0.0)
    qx = jnp.maximum(x2_ref[...] * sx + ox, 0.0)
    fy = jnp.minimum(jnp.floor(qy), float(h - 2))
    fx = jnp.minimum(jnp.floor(qx), float(w - 2))
    ay = jnp.minimum(qy - fy, 1.0)
    ax = jnp.minimum(qx - fx, 1.0)
    v1 = v1_ref[...]
    v2 = v2_ref[...]
    himask = jnp.int32(-65536)
    tl = lax.bitcast_convert_type(v1 << 16, jnp.float32)
    tr = lax.bitcast_convert_type(v1 & himask, jnp.float32)
    bl = lax.bitcast_convert_type(v2 << 16, jnp.float32)
    br = lax.bitcast_convert_type(v2 & himask, jnp.float32)
    top = ax * (tr - tl) + tl
    bot = ax * (br - bl) + bl
    out_ref[...] = ay * (bot - top) + top


def _combine(h, w, params, x1r, x2r, v1r, v2r):
    rows = x1r.shape[0]
    blk = 1024
    spec = pl.BlockSpec((blk, 128), lambda i: (i, 0))
    return pl.pallas_call(
        functools.partial(_combine_kernel, h, w),
        out_shape=jax.ShapeDtypeStruct((rows, 128), jnp.float32),
        grid=(rows // blk,),
        in_specs=[pl.BlockSpec(memory_space=pltpu.SMEM),
                  spec, spec, spec, spec],
        out_specs=spec,
    )(params, x1r, x2r, v1r, v2r)


def kernel(inputs, grid, bounds):
    n = inputs.shape[0]
    _, h, w, _ = grid.shape
    scale = (jnp.array([h, w], jnp.float32) - 1.0) / (bounds[1] - bounds[0])
    off = -bounds[0] * scale
    params = jnp.concatenate([scale, off]).astype(jnp.float32)
    rows = n // 128
    x1r = inputs[:, 0].reshape(rows, 128)
    x2r = inputs[:, 1].reshape(rows, 128)
    qa = _pack_pairs(grid.reshape(h * w // 128, 128))
    lin = _index(h, w, params, x1r, x2r)
    v1, v2 = _make_sc_gather(n, w)(lin.reshape(-1), qa.reshape(-1))
    out = _combine(h, w, params, x1r, x2r,
                   v1.reshape(rows, 128), v2.reshape(rows, 128))
    return out.reshape(n, 1)


# speedup vs baseline: 1.3670x; 1.0486x over previous
import functools

import jax
import jax.numpy as jnp
from jax import lax
from jax.experimental import pallas as pl
from jax.experimental.pallas import tpu as pltpu
from jax.experimental.pallas import tpu_sc as plsc

NC = 2
NS = 16
NW = NC * NS
L = 16
ND = 4


def _prep_kernel(h, w, params_ref, tbl_ref, x1_ref, x2_ref, qa_ref, lin_ref):
    blk = tbl_ref[...]
    lo = lax.bitcast_convert_type(blk.astype(jnp.bfloat16), jnp.uint16)
    col0_up = jnp.concatenate([lo[1:, :1], lo[:1, :1]], axis=0)
    hi = jnp.concatenate([lo[:, 1:], col0_up], axis=1)
    qa_ref[...] = lax.bitcast_convert_type(
        lo.astype(jnp.uint32) | (hi.astype(jnp.uint32) << 16), jnp.int32)

    sy = params_ref[0]
    sx = params_ref[1]
    oy = params_ref[2]
    ox = params_ref[3]
    qy = jnp.maximum(x1_ref[...] * sy + oy, 0.0)
    qx = jnp.maximum(x2_ref[...] * sx + ox, 0.0)
    fy = jnp.minimum(qy.astype(jnp.int32), h - 2)
    fx = jnp.minimum(qx.astype(jnp.int32), w - 2)
    lin_ref[...] = fy * w + fx


def _prep(h, w, params, t128, x1r, x2r):
    g = 16
    tr = t128.shape[0] // g
    xr = x1r.shape[0] // g
    xspec = pl.BlockSpec((xr, 128), lambda i: (i, 0))
    return pl.pallas_call(
        functools.partial(_prep_kernel, h, w),
        out_shape=(jax.ShapeDtypeStruct(t128.shape, jnp.int32),
                   jax.ShapeDtypeStruct((x1r.shape[0], 128), jnp.int32)),
        grid=(g,),
        in_specs=[pl.BlockSpec(memory_space=pltpu.SMEM),
                  pl.BlockSpec((tr, 128), lambda i: (i, 0)),
                  xspec, xspec],
        out_specs=(pl.BlockSpec((tr, 128), lambda i: (i, 0)), xspec),
    )(params, t128, x1r, x2r)


def _make_sc_gather(n, w):
    per_w = n // NW
    t = 2048
    nb = per_w // t
    mesh = plsc.VectorSubcoreMesh(core_axis_name="c", subcore_axis_name="s")

    ring = lambda shp, dt: [pltpu.VMEM(shp, dt) for _ in range(ND)]

    @functools.partial(
        pl.kernel,
        mesh=mesh,
        out_type=(jax.ShapeDtypeStruct((n,), jnp.int32),
                  jax.ShapeDtypeStruct((n,), jnp.int32)),
        scratch_types=(
            [pltpu.VMEM((per_w,), jnp.int32)]
            + ring((2 * t,), jnp.int32) + ring((2 * t,), jnp.int32)
            + [pltpu.SemaphoreType.DMA] * ND
        ),
    )
    def kern(lin_hbm, qa_hbm, v1_hbm, v2_hbm, lin_v, *sc):
        idxs, valss = sc[0:ND], sc[ND:2 * ND]
        sems = sc[2 * ND:3 * ND]

        cid = lax.axis_index("c")
        sid = lax.axis_index("s")
        wid = sid * NC + cid
        base_w = wid * per_w

        def build_idx(b, p):
            idx_v = idxs[p]
            base = b * t

            def body(j, carry):
                s = j * L
                v = lin_v[pl.ds(base + s, L)]
                idx_v[pl.ds(s, L)] = v
                idx_v[pl.ds(t + s, L)] = v + w
                return carry

            lax.fori_loop(0, t // L, body, 0, unroll=8)

        def start_gather(p):
            return pltpu.async_copy(qa_hbm.at[idxs[p]], valss[p], sems[p])

        def store_vals(b, p):
            off = base_w + b * t
            pltpu.sync_copy(valss[p].at[pl.ds(0, t)], v1_hbm.at[pl.ds(off, t)])
            pltpu.sync_copy(valss[p].at[pl.ds(t, t)], v2_hbm.at[pl.ds(off, t)])

        with jax.named_scope("load_lin"):
            pltpu.sync_copy(lin_hbm.at[pl.ds(base_w, per_w)], lin_v)

        handles = {}
        for b in range(nb):
            p = b % ND
            if b >= ND:
                with jax.named_scope("gather_wait"):
                    handles[b - ND].wait()
                with jax.named_scope("store_vals"):
                    store_vals(b - ND, p)
            with jax.named_scope("build_idx"):
                build_idx(b, p)
            handles[b] = start_gather(p)
        for b in range(nb - ND, nb):
            with jax.named_scope("gather_wait"):
                handles[b].wait()
            with jax.named_scope("store_vals"):
                store_vals(b, b % ND)

    return kern


def _combine_kernel(h, w, params_ref, x1_ref, x2_ref, v1_ref, v2_ref, out_ref):
    sy = params_ref[0]
    sx = params_ref[1]
    oy = params_ref[2]
    ox = params_ref[3]
    qy = jnp.maximum(x1_ref[...] * sy + oy, 0.0)
    qx = jnp.maximum(x2_ref[...] * sx + ox, 0.0)
    fy = jnp.minimum(jnp.floor(qy), float(h - 2))
    fx = jnp.minimum(jnp.floor(qx), float(w - 2))
    ay = jnp.minimum(qy - fy, 1.0)
    ax = jnp.minimum(qx - fx, 1.0)
    v1 = v1_ref[...]
    v2 = v2_ref[...]
    himask = jnp.int32(-65536)
    tl = lax.bitcast_convert_type(v1 << 16, jnp.float32)
    tr = lax.bitcast_convert_type(v1 & himask, jnp.float32)
    bl = lax.bitcast_convert_type(v2 << 16, jnp.float32)
    br = lax.bitcast_convert_type(v2 & himask, jnp.float32)
    top = ax * (tr - tl) + tl
    bot = ax * (br - bl) + bl
    out_ref[...] = ay * (bot - top) + top


def _combine(h, w, params, x1r, x2r, v1r, v2r):
    rows = x1r.shape[0]
    blk = 1024
    spec = pl.BlockSpec((blk, 128), lambda i: (i, 0))
    return pl.pallas_call(
        functools.partial(_combine_kernel, h, w),
        out_shape=jax.ShapeDtypeStruct((rows, 128), jnp.float32),
        grid=(rows // blk,),
        in_specs=[pl.BlockSpec(memory_space=pltpu.SMEM),
                  spec, spec, spec, spec],
        out_specs=spec,
    )(params, x1r, x2r, v1r, v2r)


def kernel(inputs, grid, bounds):
    n = inputs.shape[0]
    _, h, w, _ = grid.shape
    scale = (jnp.array([h, w], jnp.float32) - 1.0) / (bounds[1] - bounds[0])
    off = -bounds[0] * scale
    params = jnp.concatenate([scale, off]).astype(jnp.float32)
    rows = n // 128
    x1r = inputs[:, 0].reshape(rows, 128)
    x2r = inputs[:, 1].reshape(rows, 128)
    qa, lin = _prep(h, w, params, grid.reshape(h * w // 128, 128), x1r, x2r)
    v1, v2 = _make_sc_gather(n, w)(lin.reshape(-1), qa.reshape(-1))
    out = _combine(h, w, params, x1r, x2r,
                   v1.reshape(rows, 128), v2.reshape(rows, 128))
    return out.reshape(n, 1)
